# baseline (speedup 1.0000x reference)
import functools

import jax
import jax.numpy as jnp
from jax import lax
from jax.experimental import pallas as pl
from jax.experimental.pallas import tpu as pltpu
from jax.experimental.pallas import tpu_sc as plsc

N_NODES_C = 10000
D_C = 128
HD = D_C // 2
E_C = 320000

NUM_CORES = 2
NUM_SUBCORES = 16
CHUNK = 256
NBUF = 3
CHUNKS_PER_T = 80
E_PAD = NUM_SUBCORES * CHUNKS_PER_T * CHUNK
N_PAD = 10240
ROWS_PER_TILE = N_PAD // NUM_SUBCORES


BLK = 10
NBLK = CHUNKS_PER_T // BLK
LASTB = (BLK - 1) % NBUF
_AGG_SLICES = [CHUNK] * (ROWS_PER_TILE // CHUNK)
if ROWS_PER_TILE % CHUNK:
    _AGG_SLICES.append(ROWS_PER_TILE % CHUNK)


def _sc_aggregate_body(x_hbm, src_hbm, dst_hbm, wts_hbm, part_hbm,
                       srcb, dstb, wtsb, rows, gsems, ssems, isems,
                       agg_s):
    cid = lax.axis_index("c")
    sid = lax.axis_index("s")
    ph = part_hbm.at[cid]
    src_h = src_hbm.at[sid]
    dst_h = dst_hbm.at[sid]
    wts_h = wts_hbm.at[sid]

    def scale_rows(buf, slot):
        def group_body(g, _):
            wv = wtsb[slot, pl.ds(g * 16, 16)]
            for e16 in range(16):
                e = g * 16 + e16
                w = wv[e16]
                for k in range(HD // 16):
                    sl = pl.ds(k * 16, 16)
                    buf[e, sl] = buf[e, sl] * w
            return 0

        lax.fori_loop(0, CHUNK // 16, group_body, 0)

    zeros16 = jnp.zeros((16,), jnp.float32)

    def zrow(i, _):
        for k in range(HD // 16):
            rows[0][i, pl.ds(k * 16, 16)] = zeros16
        return 0

    lax.fori_loop(0, CHUNK, zrow, 0)

    base = sid * ROWS_PER_TILE
    off = 0
    for sz in _AGG_SLICES:
        pltpu.sync_copy(rows[0].at[pl.ds(0, sz)],
                        agg_s.at[pl.ds(base + off, sz)])
        off += sz

    xh = x_hbm.at[cid]

    sl0 = pl.ds(0, BLK)
    pltpu.async_copy(src_h.at[sl0], srcb.at[sl0], isems[0])
    pltpu.async_copy(dst_h.at[sl0], dstb.at[sl0], isems[0])
    pltpu.async_copy(wts_h.at[sl0], wtsb.at[sl0], isems[0])

    plsc.subcore_barrier()

    def block_pair_body(q2, _):
      for par in range(2):
        q = q2 * 2 + par
        roff = par * BLK
        hoff = q * BLK
        for a, (h, r) in enumerate(((src_h, srcb), (dst_h, dstb),
                                    (wts_h, wtsb))):
            pltpu.make_async_copy(
                h.at[pl.ds(hoff, BLK)], r.at[pl.ds(roff, BLK)],
                isems[par]).wait()

        @pl.when(q > 0)
        def _():
            pltpu.make_async_copy(
                rows[LASTB], agg_s.at[dstb.at[BLK - 1 + (1 - par) * BLK]],
                ssems[LASTB]).wait()

        @pl.when(q + 1 < NBLK)
        def _():
            nroff = (1 - par) * BLK
            nhoff = hoff + BLK
            pltpu.async_copy(src_h.at[pl.ds(nhoff, BLK)],
                             srcb.at[pl.ds(nroff, BLK)], isems[1 - par])
            pltpu.async_copy(dst_h.at[pl.ds(nhoff, BLK)],
                             dstb.at[pl.ds(nroff, BLK)], isems[1 - par])
            pltpu.async_copy(wts_h.at[pl.ds(nhoff, BLK)],
                             wtsb.at[pl.ds(nroff, BLK)], isems[1 - par])

        for b in range(NBUF):
            pltpu.async_copy(xh.at[srcb.at[roff + b]], rows[b], gsems[b])

        for t in range(BLK):
            b = t % NBUF
            bp = (t + NBUF - 1) % NBUF
            slot = roff + t
            pltpu.make_async_copy(
                xh.at[srcb.at[slot]], rows[b], gsems[b]).wait()

            if t >= 1:
                pltpu.make_async_copy(
                    rows[bp], agg_s.at[dstb.at[slot - 1]],
                    ssems[bp]).wait()
                if t + NBUF - 1 < BLK:
                    pltpu.async_copy(
                        xh.at[srcb.at[slot + NBUF - 1]], rows[bp],
                        gsems[bp])

            scale_rows(rows[b], slot)
            pltpu.async_copy(rows[b], agg_s.at[dstb.at[slot]], ssems[b],
                             add=True)
      return 0

    lax.fori_loop(0, NBLK // 2, block_pair_body, 0)

    pltpu.make_async_copy(
        rows[LASTB], agg_s.at[dstb.at[BLK - 1 + ((NBLK - 1) % 2) * BLK]],
        ssems[LASTB]).wait()

    plsc.subcore_barrier()

    off = 0
    for sz in _AGG_SLICES:
        pltpu.sync_copy(agg_s.at[pl.ds(base + off, sz)],
                        ph.at[pl.ds(base + off, sz)])
        off += sz


@functools.partial(
    pl.kernel,
    out_type=jax.ShapeDtypeStruct((NUM_CORES, N_PAD, HD), jnp.float32),
    mesh=plsc.VectorSubcoreMesh(core_axis_name="c", subcore_axis_name="s"),
    compiler_params=pltpu.CompilerParams(use_tc_tiling_on_sc=False),
    scratch_types=[
        pltpu.VMEM((2 * BLK, CHUNK), jnp.int32),
        pltpu.VMEM((2 * BLK, CHUNK), jnp.int32),
        pltpu.VMEM((2 * BLK, CHUNK), jnp.float32),
        [pltpu.VMEM((CHUNK, HD), jnp.float32)] * NBUF,
        [pltpu.SemaphoreType.DMA] * NBUF,
        [pltpu.SemaphoreType.DMA] * NBUF,
        [pltpu.SemaphoreType.DMA] * 2,
        pltpu.VMEM_SHARED((N_PAD, HD), jnp.float32),
    ],
)
def _sc_aggregate(x_hbm, src_hbm, dst_hbm, wts_hbm, part_hbm,
                  srcb, dstb, wtsb, rows, gsems, ssems, isems, agg_s):
    _sc_aggregate_body(x_hbm, src_hbm, dst_hbm, wts_hbm, part_hbm,
                       srcb, dstb, wtsb, rows, gsems, ssems, isems,
                       agg_s)


def _tc_layer_body(p_ref, x_ref, w_ref, b_ref, o_ref):
    s = jnp.concatenate(
        [p_ref[0] + x_ref[0], p_ref[1] + x_ref[1]], axis=1)
    y = lax.dot_general(s, w_ref[...], (((1,), (1,)), ((), ())),
                        preferred_element_type=jnp.float32)
    y = jnp.maximum(y + b_ref[...], 0.0)
    o_ref[0] = y[:, :HD]
    o_ref[1] = y[:, HD:]


def _tc_final_body(p_ref, x_ref, w1_ref, b1_ref, wc1_ref, bc1_ref,
                   wc2_ref, bc2_ref, o_ref):
    s = jnp.concatenate(
        [p_ref[0] + x_ref[0], p_ref[1] + x_ref[1]], axis=1)
    x2 = lax.dot_general(s, w1_ref[...], (((1,), (1,)), ((), ())),
                         preferred_element_type=jnp.float32)
    x2 = jnp.maximum(x2 + b1_ref[...], 0.0)
    h = lax.dot_general(x2, wc1_ref[...], (((1,), (1,)), ((), ())),
                        preferred_element_type=jnp.float32)
    h = jnp.maximum(h + bc1_ref[...], 0.0)
    logits = lax.dot_general(h, wc2_ref[...], (((1,), (1,)), ((), ())),
                             preferred_element_type=jnp.float32)
    o_ref[...] = logits + bc2_ref[...]


_ROW_BLK = 2000


def _tc_layer(part, x, W, b):
    grid = (N_NODES_C // _ROW_BLK,)
    return pl.pallas_call(
        _tc_layer_body,
        grid=grid,
        in_specs=[
            pl.BlockSpec((NUM_CORES, _ROW_BLK, HD), lambda r: (0, r, 0)),
            pl.BlockSpec((NUM_CORES, _ROW_BLK, HD), lambda r: (0, r, 0)),
            pl.BlockSpec((D_C, D_C), lambda r: (0, 0)),
            pl.BlockSpec((1, D_C), lambda r: (0, 0)),
        ],
        out_specs=pl.BlockSpec((NUM_CORES, _ROW_BLK, HD), lambda r: (0, r, 0)),
        out_shape=jax.ShapeDtypeStruct((NUM_CORES, N_NODES_C, HD),
                                       jnp.float32),
    )(part, x, W, b)


def _tc_final(part, x, W1, b1, Wc1p, bc1p, Wc2p, bc2p):
    grid = (N_NODES_C // _ROW_BLK,)
    return pl.pallas_call(
        _tc_final_body,
        grid=grid,
        in_specs=[
            pl.BlockSpec((NUM_CORES, _ROW_BLK, HD), lambda r: (0, r, 0)),
            pl.BlockSpec((NUM_CORES, _ROW_BLK, HD), lambda r: (0, r, 0)),
            pl.BlockSpec((D_C, D_C), lambda r: (0, 0)),
            pl.BlockSpec((1, D_C), lambda r: (0, 0)),
            pl.BlockSpec((D_C, D_C), lambda r: (0, 0)),
            pl.BlockSpec((1, D_C), lambda r: (0, 0)),
            pl.BlockSpec((D_C, D_C), lambda r: (0, 0)),
            pl.BlockSpec((1, D_C), lambda r: (0, 0)),
        ],
        out_specs=pl.BlockSpec((_ROW_BLK, D_C), lambda r: (r, 0)),
        out_shape=jax.ShapeDtypeStruct((N_NODES_C, D_C), jnp.float32),
    )(part, x, W1, b1, Wc1p, bc1p, Wc2p, bc2p)


def kernel(features, edge_indices, edge_weights, W0, b0, W1, b1,
           Wc1, bc1, Wc2, bc2):
    ei = edge_indices[0].astype(jnp.int32)
    ew = edge_weights[0]
    pad = E_PAD - E_C
    src = jnp.concatenate([ei[0], jnp.zeros((pad,), jnp.int32)])
    dst = jnp.concatenate([ei[1], jnp.zeros((pad,), jnp.int32)])
    wts = jnp.concatenate([ew, jnp.zeros((pad,), jnp.float32)])
    src = src.reshape(NUM_SUBCORES, CHUNKS_PER_T, CHUNK)
    dst = dst.reshape(NUM_SUBCORES, CHUNKS_PER_T, CHUNK)
    wts = wts.reshape(NUM_SUBCORES, CHUNKS_PER_T, CHUNK)

    H2 = D_C // 2
    Wc1p = jnp.zeros((D_C, D_C), jnp.float32).at[:H2, :].set(Wc1)
    bc1p = jnp.zeros((1, D_C), jnp.float32).at[0, :H2].set(bc1)
    Wc2p = jnp.zeros((D_C, D_C), jnp.float32).at[:2, :H2].set(Wc2)
    bc2p = jnp.zeros((1, D_C), jnp.float32).at[0, :2].set(bc2)

    b0r = b0.reshape(1, D_C)
    b1r = b1.reshape(1, D_C)

    x = jnp.stack([features[:, :HD], features[:, HD:]])
    part0 = _sc_aggregate(x, src, dst, wts)
    x1 = _tc_layer(part0, x, W0, b0r)
    part1 = _sc_aggregate(x1, src, dst, wts)
    out = _tc_final(part1, x1, W1, b1r, Wc1p, bc1p, Wc2p, bc2p)
    return out[:, :2]


# speedup vs baseline: 1.0895x; 1.0895x over previous
import functools

import jax
import jax.numpy as jnp
from jax import lax
from jax.experimental import pallas as pl
from jax.experimental.pallas import tpu as pltpu
from jax.experimental.pallas import tpu_sc as plsc

N_NODES_C = 10000
D_C = 128
HD = D_C // 2
E_C = 320000

NUM_CORES = 2
NUM_SUBCORES = 16
NW = NUM_CORES * NUM_SUBCORES
CHUNK = 128
NBUF = 2
CHUNKS_PER_T = 80
E_PAD = NW * CHUNKS_PER_T * CHUNK
N_PAD = 10240
ROWS_PER_TILE = N_PAD // NUM_SUBCORES


BLK = 10
NBLK = CHUNKS_PER_T // BLK
LASTB = (BLK - 1) % NBUF
_AGG_SLICES = [CHUNK] * (ROWS_PER_TILE // CHUNK)
if ROWS_PER_TILE % CHUNK:
    _AGG_SLICES.append(ROWS_PER_TILE % CHUNK)


def _sc_aggregate_body(x_hbm, src_hbm, dst_hbm, wts_hbm, part_hbm,
                       srcb, dstb, wtsb, rows, gsems, ssems, isems,
                       agg_s):
    cid = lax.axis_index("c")
    sid = lax.axis_index("s")
    wid = cid * NUM_SUBCORES + sid
    ph = part_hbm.at[cid]
    src_h = src_hbm.at[wid]
    dst_h = dst_hbm.at[wid]
    wts_h = wts_hbm.at[wid]

    def scale_rows(buf, slot):
        def group_body(g, _):
            wv = wtsb[slot, pl.ds(g * 16, 16)]
            for e16 in range(16):
                e = g * 16 + e16
                w = wv[e16]
                for k in range(D_C // 16):
                    sl = pl.ds(k * 16, 16)
                    buf[e, sl] = buf[e, sl] * w
            return 0

        lax.fori_loop(0, CHUNK // 16, group_body, 0)

    zeros16 = jnp.zeros((16,), jnp.float32)

    def zrow(i, _):
        for k in range(D_C // 16):
            rows[0][i, pl.ds(k * 16, 16)] = zeros16
        return 0

    lax.fori_loop(0, CHUNK, zrow, 0)

    base = sid * ROWS_PER_TILE
    off = 0
    for sz in _AGG_SLICES:
        pltpu.sync_copy(rows[0].at[pl.ds(0, sz)],
                        agg_s.at[pl.ds(base + off, sz)])
        off += sz

    xh = x_hbm

    sl0 = pl.ds(0, BLK)
    pltpu.async_copy(src_h.at[sl0], srcb.at[sl0], isems[0])
    pltpu.async_copy(dst_h.at[sl0], dstb.at[sl0], isems[0])
    pltpu.async_copy(wts_h.at[sl0], wtsb.at[sl0], isems[0])

    plsc.subcore_barrier()

    def block_pair_body(q2, _):
      for par in range(2):
        q = q2 * 2 + par
        roff = par * BLK
        hoff = q * BLK
        for a, (h, r) in enumerate(((src_h, srcb), (dst_h, dstb),
                                    (wts_h, wtsb))):
            pltpu.make_async_copy(
                h.at[pl.ds(hoff, BLK)], r.at[pl.ds(roff, BLK)],
                isems[par]).wait()

        @pl.when(q > 0)
        def _():
            pltpu.make_async_copy(
                rows[LASTB], agg_s.at[dstb.at[BLK - 1 + (1 - par) * BLK]],
                ssems[LASTB]).wait()

        @pl.when(q + 1 < NBLK)
        def _():
            nroff = (1 - par) * BLK
            nhoff = hoff + BLK
            pltpu.async_copy(src_h.at[pl.ds(nhoff, BLK)],
                             srcb.at[pl.ds(nroff, BLK)], isems[1 - par])
            pltpu.async_copy(dst_h.at[pl.ds(nhoff, BLK)],
                             dstb.at[pl.ds(nroff, BLK)], isems[1 - par])
            pltpu.async_copy(wts_h.at[pl.ds(nhoff, BLK)],
                             wtsb.at[pl.ds(nroff, BLK)], isems[1 - par])

        for b in range(NBUF):
            pltpu.async_copy(xh.at[srcb.at[roff + b]], rows[b], gsems[b])

        for t in range(BLK):
            b = t % NBUF
            bp = (t + NBUF - 1) % NBUF
            slot = roff + t
            pltpu.make_async_copy(
                xh.at[srcb.at[slot]], rows[b], gsems[b]).wait()

            if t >= 1:
                pltpu.make_async_copy(
                    rows[bp], agg_s.at[dstb.at[slot - 1]],
                    ssems[bp]).wait()
                if t + NBUF - 1 < BLK:
                    pltpu.async_copy(
                        xh.at[srcb.at[slot + NBUF - 1]], rows[bp],
                        gsems[bp])

            scale_rows(rows[b], slot)
            pltpu.async_copy(rows[b], agg_s.at[dstb.at[slot]], ssems[b],
                             add=True)
      return 0

    lax.fori_loop(0, NBLK // 2, block_pair_body, 0)

    pltpu.make_async_copy(
        rows[LASTB], agg_s.at[dstb.at[BLK - 1 + ((NBLK - 1) % 2) * BLK]],
        ssems[LASTB]).wait()

    plsc.subcore_barrier()

    off = 0
    for sz in _AGG_SLICES:
        pltpu.sync_copy(agg_s.at[pl.ds(base + off, sz)],
                        ph.at[pl.ds(base + off, sz)])
        off += sz


@functools.partial(
    pl.kernel,
    out_type=jax.ShapeDtypeStruct((NUM_CORES, N_PAD, D_C), jnp.float32),
    mesh=plsc.VectorSubcoreMesh(core_axis_name="c", subcore_axis_name="s"),
    compiler_params=pltpu.CompilerParams(use_tc_tiling_on_sc=False),
    scratch_types=[
        pltpu.VMEM((2 * BLK, CHUNK), jnp.int32),
        pltpu.VMEM((2 * BLK, CHUNK), jnp.int32),
        pltpu.VMEM((2 * BLK, CHUNK), jnp.float32),
        [pltpu.VMEM((CHUNK, D_C), jnp.float32)] * NBUF,
        [pltpu.SemaphoreType.DMA] * NBUF,
        [pltpu.SemaphoreType.DMA] * NBUF,
        [pltpu.SemaphoreType.DMA] * 2,
        pltpu.VMEM_SHARED((N_PAD, D_C), jnp.float32),
    ],
)
def _sc_aggregate(x_hbm, src_hbm, dst_hbm, wts_hbm, part_hbm,
                  srcb, dstb, wtsb, rows, gsems, ssems, isems, agg_s):
    _sc_aggregate_body(x_hbm, src_hbm, dst_hbm, wts_hbm, part_hbm,
                       srcb, dstb, wtsb, rows, gsems, ssems, isems,
                       agg_s)


def _tc_layer_body(p_ref, x_ref, w_ref, b_ref, o_ref):
    s = p_ref[0] + p_ref[1] + x_ref[...]
    y = lax.dot_general(s, w_ref[...], (((1,), (1,)), ((), ())),
                        preferred_element_type=jnp.float32)
    o_ref[...] = jnp.maximum(y + b_ref[...], 0.0)


def _tc_final_body(p_ref, x_ref, w1_ref, b1_ref, wc1_ref, bc1_ref,
                   wc2_ref, bc2_ref, o_ref):
    s = p_ref[0] + p_ref[1] + x_ref[...]
    x2 = lax.dot_general(s, w1_ref[...], (((1,), (1,)), ((), ())),
                         preferred_element_type=jnp.float32)
    x2 = jnp.maximum(x2 + b1_ref[...], 0.0)
    h = lax.dot_general(x2, wc1_ref[...], (((1,), (1,)), ((), ())),
                        preferred_element_type=jnp.float32)
    h = jnp.maximum(h + bc1_ref[...], 0.0)
    logits = lax.dot_general(h, wc2_ref[...], (((1,), (1,)), ((), ())),
                             preferred_element_type=jnp.float32)
    o_ref[...] = logits + bc2_ref[...]


_ROW_BLK = 2000


def _tc_layer(part, x, W, b):
    grid = (N_NODES_C // _ROW_BLK,)
    return pl.pallas_call(
        _tc_layer_body,
        grid=grid,
        in_specs=[
            pl.BlockSpec((NUM_CORES, _ROW_BLK, D_C), lambda r: (0, r, 0)),
            pl.BlockSpec((_ROW_BLK, D_C), lambda r: (r, 0)),
            pl.BlockSpec((D_C, D_C), lambda r: (0, 0)),
            pl.BlockSpec((1, D_C), lambda r: (0, 0)),
        ],
        out_specs=pl.BlockSpec((_ROW_BLK, D_C), lambda r: (r, 0)),
        out_shape=jax.ShapeDtypeStruct((N_NODES_C, D_C), jnp.float32),
    )(part, x, W, b)


def _tc_final(part, x, W1, b1, Wc1p, bc1p, Wc2p, bc2p):
    grid = (N_NODES_C // _ROW_BLK,)
    return pl.pallas_call(
        _tc_final_body,
        grid=grid,
        in_specs=[
            pl.BlockSpec((NUM_CORES, _ROW_BLK, D_C), lambda r: (0, r, 0)),
            pl.BlockSpec((_ROW_BLK, D_C), lambda r: (r, 0)),
            pl.BlockSpec((D_C, D_C), lambda r: (0, 0)),
            pl.BlockSpec((1, D_C), lambda r: (0, 0)),
            pl.BlockSpec((D_C, D_C), lambda r: (0, 0)),
            pl.BlockSpec((1, D_C), lambda r: (0, 0)),
            pl.BlockSpec((D_C, D_C), lambda r: (0, 0)),
            pl.BlockSpec((1, D_C), lambda r: (0, 0)),
        ],
        out_specs=pl.BlockSpec((_ROW_BLK, D_C), lambda r: (r, 0)),
        out_shape=jax.ShapeDtypeStruct((N_NODES_C, D_C), jnp.float32),
    )(part, x, W1, b1, Wc1p, bc1p, Wc2p, bc2p)


def kernel(features, edge_indices, edge_weights, W0, b0, W1, b1,
           Wc1, bc1, Wc2, bc2):
    ei = edge_indices[0].astype(jnp.int32)
    ew = edge_weights[0]
    pad = E_PAD - E_C
    src = jnp.concatenate([ei[0], jnp.zeros((pad,), jnp.int32)])
    dst = jnp.concatenate([ei[1], jnp.zeros((pad,), jnp.int32)])
    wts = jnp.concatenate([ew, jnp.zeros((pad,), jnp.float32)])
    src = src.reshape(NW, CHUNKS_PER_T, CHUNK)
    dst = dst.reshape(NW, CHUNKS_PER_T, CHUNK)
    wts = wts.reshape(NW, CHUNKS_PER_T, CHUNK)

    H2 = D_C // 2
    Wc1p = jnp.zeros((D_C, D_C), jnp.float32).at[:H2, :].set(Wc1)
    bc1p = jnp.zeros((1, D_C), jnp.float32).at[0, :H2].set(bc1)
    Wc2p = jnp.zeros((D_C, D_C), jnp.float32).at[:2, :H2].set(Wc2)
    bc2p = jnp.zeros((1, D_C), jnp.float32).at[0, :2].set(bc2)

    b0r = b0.reshape(1, D_C)
    b1r = b1.reshape(1, D_C)

    x = features
    part0 = _sc_aggregate(x, src, dst, wts)
    x1 = _tc_layer(part0, x, W0, b0r)
    part1 = _sc_aggregate(x1, src, dst, wts)
    out = _tc_final(part1, x1, W1, b1r, Wc1p, bc1p, Wc2p, bc2p)
    return out[:, :2]


# speedup vs baseline: 1.1077x; 1.0167x over previous
import functools

import jax
import jax.numpy as jnp
from jax import lax
from jax.experimental import pallas as pl
from jax.experimental.pallas import tpu as pltpu
from jax.experimental.pallas import tpu_sc as plsc

N_NODES_C = 10000
D_C = 128
HD = D_C // 2
E_C = 320000

NUM_CORES = 2
NUM_SUBCORES = 16
CHUNK = 128
NBUF = 5
CHUNKS_PER_T = 162
E_PAD = NUM_SUBCORES * CHUNKS_PER_T * CHUNK
N_PAD = 10240
ROWS_PER_TILE = N_PAD // NUM_SUBCORES

BLK = 9
NBLK = CHUNKS_PER_T // BLK
LASTB = (BLK - 1) % NBUF
_AGG_SLICES = [CHUNK] * (ROWS_PER_TILE // CHUNK)
if ROWS_PER_TILE % CHUNK:
    _AGG_SLICES.append(ROWS_PER_TILE % CHUNK)


def _sc_aggregate_body(x_hbm, src_hbm, dst_hbm, wts_hbm, part_hbm,
                       srcb, dstb, wtsb, rows, gsems, ssems, isems,
                       xs, agg_s):
    cid = lax.axis_index("c")
    sid = lax.axis_index("s")
    ph = part_hbm.at[cid]
    src_h = src_hbm.at[sid]
    dst_h = dst_hbm.at[sid]
    wts_h = wts_hbm.at[sid]

    def scale_rows(buf, slot):
        def group_body(g, _):
            wv = wtsb[slot, pl.ds(g * 16, 16)]
            for e16 in range(16):
                e = g * 16 + e16
                w = wv[e16]
                for k in range(HD // 16):
                    sl = pl.ds(k * 16, 16)
                    buf[e, sl] = buf[e, sl] * w
            return 0

        lax.fori_loop(0, CHUNK // 16, group_body, 0)

    zeros16 = jnp.zeros((16,), jnp.float32)

    def zrow(i, _):
        for k in range(HD // 16):
            rows[0][i, pl.ds(k * 16, 16)] = zeros16
        return 0

    lax.fori_loop(0, CHUNK, zrow, 0)

    base = sid * ROWS_PER_TILE
    off = 0
    for sz in _AGG_SLICES:
        pltpu.sync_copy(rows[0].at[pl.ds(0, sz)],
                        agg_s.at[pl.ds(base + off, sz)])
        off += sz

    xh = x_hbm.at[cid]

    @pl.when(sid < NUM_SUBCORES - 1)
    def _():
        pltpu.sync_copy(xh.at[pl.ds(base, ROWS_PER_TILE)],
                        xs.at[pl.ds(base, ROWS_PER_TILE)])

    @pl.when(sid == NUM_SUBCORES - 1)
    def _():
        lo = (NUM_SUBCORES - 1) * ROWS_PER_TILE
        pltpu.sync_copy(xh.at[pl.ds(lo, N_NODES_C - lo)],
                        xs.at[pl.ds(lo, N_NODES_C - lo)])

    sl0 = pl.ds(0, BLK)
    pltpu.async_copy(src_h.at[sl0], srcb.at[sl0], isems[0])
    pltpu.async_copy(dst_h.at[sl0], dstb.at[sl0], isems[0])
    pltpu.async_copy(wts_h.at[sl0], wtsb.at[sl0], isems[0])

    plsc.subcore_barrier()

    def block_pair_body(q2, _):
      for par in range(2):
        q = q2 * 2 + par
        roff = par * BLK
        hoff = q * BLK
        for h, r in ((src_h, srcb), (dst_h, dstb), (wts_h, wtsb)):
            pltpu.make_async_copy(
                h.at[pl.ds(hoff, BLK)], r.at[pl.ds(roff, BLK)],
                isems[par]).wait()

        @pl.when(q > 0)
        def _():
            pltpu.make_async_copy(
                rows[LASTB], agg_s.at[dstb.at[BLK - 1 + (1 - par) * BLK]],
                ssems[LASTB]).wait()

        @pl.when(q + 1 < NBLK)
        def _():
            nroff = (1 - par) * BLK
            nhoff = hoff + BLK
            pltpu.async_copy(src_h.at[pl.ds(nhoff, BLK)],
                             srcb.at[pl.ds(nroff, BLK)], isems[1 - par])
            pltpu.async_copy(dst_h.at[pl.ds(nhoff, BLK)],
                             dstb.at[pl.ds(nroff, BLK)], isems[1 - par])
            pltpu.async_copy(wts_h.at[pl.ds(nhoff, BLK)],
                             wtsb.at[pl.ds(nroff, BLK)], isems[1 - par])

        for b in range(min(NBUF, BLK)):
            pltpu.async_copy(xs.at[srcb.at[roff + b]], rows[b], gsems[b])

        for t in range(BLK):
            b = t % NBUF
            bp = (t + NBUF - 1) % NBUF
            slot = roff + t
            pltpu.make_async_copy(
                xs.at[srcb.at[slot]], rows[b], gsems[b]).wait()

            if t >= 1:
                pltpu.make_async_copy(
                    rows[bp], agg_s.at[dstb.at[slot - 1]],
                    ssems[bp]).wait()
                if t + NBUF - 1 < BLK:
                    pltpu.async_copy(
                        xs.at[srcb.at[slot + NBUF - 1]], rows[bp],
                        gsems[bp])

            scale_rows(rows[b], slot)
            pltpu.async_copy(rows[b], agg_s.at[dstb.at[slot]], ssems[b],
                             add=True)
      return 0

    lax.fori_loop(0, NBLK // 2, block_pair_body, 0)

    pltpu.make_async_copy(
        rows[LASTB], agg_s.at[dstb.at[BLK - 1 + ((NBLK - 1) % 2) * BLK]],
        ssems[LASTB]).wait()

    plsc.subcore_barrier()

    off = 0
    for sz in _AGG_SLICES:
        pltpu.sync_copy(agg_s.at[pl.ds(base + off, sz)],
                        ph.at[pl.ds(base + off, sz)])
        off += sz


@functools.partial(
    pl.kernel,
    out_type=jax.ShapeDtypeStruct((NUM_CORES, N_PAD, HD), jnp.float32),
    mesh=plsc.VectorSubcoreMesh(core_axis_name="c", subcore_axis_name="s"),
    compiler_params=pltpu.CompilerParams(use_tc_tiling_on_sc=False),
    scratch_types=[
        pltpu.VMEM((2 * BLK, CHUNK), jnp.int32),
        pltpu.VMEM((2 * BLK, CHUNK), jnp.int32),
        pltpu.VMEM((2 * BLK, CHUNK), jnp.float32),
        [pltpu.VMEM((CHUNK, HD), jnp.float32)] * NBUF,
        [pltpu.SemaphoreType.DMA] * NBUF,
        [pltpu.SemaphoreType.DMA] * NBUF,
        [pltpu.SemaphoreType.DMA] * 2,
        pltpu.VMEM_SHARED((N_PAD, HD), jnp.float32),
        pltpu.VMEM_SHARED((N_PAD, HD), jnp.float32),
    ],
)
def _sc_aggregate(x_hbm, src_hbm, dst_hbm, wts_hbm, part_hbm,
                  srcb, dstb, wtsb, rows, gsems, ssems, isems, xs, agg_s):
    _sc_aggregate_body(x_hbm, src_hbm, dst_hbm, wts_hbm, part_hbm,
                       srcb, dstb, wtsb, rows, gsems, ssems, isems,
                       xs, agg_s)


def _tc_layer_body(p_ref, x_ref, w_ref, b_ref, o_ref):
    s = jnp.concatenate(
        [p_ref[0] + x_ref[0], p_ref[1] + x_ref[1]], axis=1)
    y = lax.dot_general(s, w_ref[...], (((1,), (1,)), ((), ())),
                        preferred_element_type=jnp.float32)
    y = jnp.maximum(y + b_ref[...], 0.0)
    o_ref[0] = y[:, :HD]
    o_ref[1] = y[:, HD:]


def _tc_final_body(p_ref, x_ref, w1_ref, b1_ref, wc1_ref, bc1_ref,
                   wc2_ref, bc2_ref, o_ref):
    s = jnp.concatenate(
        [p_ref[0] + x_ref[0], p_ref[1] + x_ref[1]], axis=1)
    x2 = lax.dot_general(s, w1_ref[...], (((1,), (1,)), ((), ())),
                         preferred_element_type=jnp.float32)
    x2 = jnp.maximum(x2 + b1_ref[...], 0.0)
    h = lax.dot_general(x2, wc1_ref[...], (((1,), (1,)), ((), ())),
                        preferred_element_type=jnp.float32)
    h = jnp.maximum(h + bc1_ref[...], 0.0)
    logits = lax.dot_general(h, wc2_ref[...], (((1,), (1,)), ((), ())),
                             preferred_element_type=jnp.float32)
    o_ref[...] = logits + bc2_ref[...]


_ROW_BLK = 2000


def _tc_layer(part, x, W, b):
    grid = (N_NODES_C // _ROW_BLK,)
    return pl.pallas_call(
        _tc_layer_body,
        grid=grid,
        in_specs=[
            pl.BlockSpec((NUM_CORES, _ROW_BLK, HD), lambda r: (0, r, 0)),
            pl.BlockSpec((NUM_CORES, _ROW_BLK, HD), lambda r: (0, r, 0)),
            pl.BlockSpec((D_C, D_C), lambda r: (0, 0)),
            pl.BlockSpec((1, D_C), lambda r: (0, 0)),
        ],
        out_specs=pl.BlockSpec((NUM_CORES, _ROW_BLK, HD), lambda r: (0, r, 0)),
        out_shape=jax.ShapeDtypeStruct((NUM_CORES, N_NODES_C, HD),
                                       jnp.float32),
    )(part, x, W, b)


def _tc_final(part, x, W1, b1, Wc1p, bc1p, Wc2p, bc2p):
    grid = (N_NODES_C // _ROW_BLK,)
    return pl.pallas_call(
        _tc_final_body,
        grid=grid,
        in_specs=[
            pl.BlockSpec((NUM_CORES, _ROW_BLK, HD), lambda r: (0, r, 0)),
            pl.BlockSpec((NUM_CORES, _ROW_BLK, HD), lambda r: (0, r, 0)),
            pl.BlockSpec((D_C, D_C), lambda r: (0, 0)),
            pl.BlockSpec((1, D_C), lambda r: (0, 0)),
            pl.BlockSpec((D_C, D_C), lambda r: (0, 0)),
            pl.BlockSpec((1, D_C), lambda r: (0, 0)),
            pl.BlockSpec((D_C, D_C), lambda r: (0, 0)),
            pl.BlockSpec((1, D_C), lambda r: (0, 0)),
        ],
        out_specs=pl.BlockSpec((_ROW_BLK, D_C), lambda r: (r, 0)),
        out_shape=jax.ShapeDtypeStruct((N_NODES_C, D_C), jnp.float32),
    )(part, x, W1, b1, Wc1p, bc1p, Wc2p, bc2p)


def kernel(features, edge_indices, edge_weights, W0, b0, W1, b1,
           Wc1, bc1, Wc2, bc2):
    ei = edge_indices[0].astype(jnp.int32)
    ew = edge_weights[0]
    pad = E_PAD - E_C
    src = jnp.concatenate([ei[0], jnp.zeros((pad,), jnp.int32)])
    dst = jnp.concatenate([ei[1], jnp.zeros((pad,), jnp.int32)])
    wts = jnp.concatenate([ew, jnp.zeros((pad,), jnp.float32)])
    src = src.reshape(NUM_SUBCORES, CHUNKS_PER_T, CHUNK)
    dst = dst.reshape(NUM_SUBCORES, CHUNKS_PER_T, CHUNK)
    wts = wts.reshape(NUM_SUBCORES, CHUNKS_PER_T, CHUNK)

    H2 = D_C // 2
    Wc1p = jnp.zeros((D_C, D_C), jnp.float32).at[:H2, :].set(Wc1)
    bc1p = jnp.zeros((1, D_C), jnp.float32).at[0, :H2].set(bc1)
    Wc2p = jnp.zeros((D_C, D_C), jnp.float32).at[:2, :H2].set(Wc2)
    bc2p = jnp.zeros((1, D_C), jnp.float32).at[0, :2].set(bc2)

    b0r = b0.reshape(1, D_C)
    b1r = b1.reshape(1, D_C)

    x = jnp.stack([features[:, :HD], features[:, HD:]])
    part0 = _sc_aggregate(x, src, dst, wts)
    x1 = _tc_layer(part0, x, W0, b0r)
    part1 = _sc_aggregate(x1, src, dst, wts)
    out = _tc_final(part1, x1, W1, b1r, Wc1p, bc1p, Wc2p, bc2p)
    return out[:, :2]


# speedup vs baseline: 1.1204x; 1.0114x over previous
import functools

import jax
import jax.numpy as jnp
from jax import lax
from jax.experimental import pallas as pl
from jax.experimental.pallas import tpu as pltpu
from jax.experimental.pallas import tpu_sc as plsc

N_NODES_C = 10000
D_C = 128
HD = D_C // 2
E_C = 320000

NUM_CORES = 2
NUM_SUBCORES = 16
CHUNK = 128
NBUF = 3
CHUNKS_PER_T = 162
E_PAD = NUM_SUBCORES * CHUNKS_PER_T * CHUNK
N_PAD = 10240
ROWS_PER_TILE = N_PAD // NUM_SUBCORES

BLK = 9
NBLK = CHUNKS_PER_T // BLK
LASTB = (BLK - 1) % NBUF
_AGG_SLICES = [CHUNK] * (ROWS_PER_TILE // CHUNK)
if ROWS_PER_TILE % CHUNK:
    _AGG_SLICES.append(ROWS_PER_TILE % CHUNK)


def _sc_aggregate_body(x_hbm, src_hbm, dst_hbm, wts_hbm, part_hbm,
                       srcb, dstb, wtsb, rows, gsems, ssems, isems,
                       xs, agg_s):
    cid = lax.axis_index("c")
    sid = lax.axis_index("s")
    ph = part_hbm.at[cid]
    src_h = src_hbm.at[sid]
    dst_h = dst_hbm.at[sid]
    wts_h = wts_hbm.at[sid]

    def scale_rows(buf, slot):
        def group_body(g, _):
            wv = wtsb[slot, pl.ds(g * 16, 16)]
            for e16 in range(16):
                e = g * 16 + e16
                w = wv[e16]
                for k in range(HD // 16):
                    sl = pl.ds(k * 16, 16)
                    buf[e, sl] = buf[e, sl] * w
            return 0

        lax.fori_loop(0, CHUNK // 16, group_body, 0)

    zeros16 = jnp.zeros((16,), jnp.float32)

    def zrow(i, _):
        for k in range(HD // 16):
            rows[0][i, pl.ds(k * 16, 16)] = zeros16
        return 0

    lax.fori_loop(0, CHUNK, zrow, 0)

    base = sid * ROWS_PER_TILE
    off = 0
    for sz in _AGG_SLICES:
        pltpu.sync_copy(rows[0].at[pl.ds(0, sz)],
                        agg_s.at[pl.ds(base + off, sz)])
        off += sz

    xh = x_hbm.at[cid]

    @pl.when(sid < NUM_SUBCORES - 1)
    def _():
        pltpu.sync_copy(xh.at[pl.ds(base, ROWS_PER_TILE)],
                        xs.at[pl.ds(base, ROWS_PER_TILE)])

    @pl.when(sid == NUM_SUBCORES - 1)
    def _():
        lo = (NUM_SUBCORES - 1) * ROWS_PER_TILE
        pltpu.sync_copy(xh.at[pl.ds(lo, N_NODES_C - lo)],
                        xs.at[pl.ds(lo, N_NODES_C - lo)])

    sl0 = pl.ds(0, BLK)
    pltpu.async_copy(src_h.at[sl0], srcb.at[sl0], isems[0])
    pltpu.async_copy(dst_h.at[sl0], dstb.at[sl0], isems[0])
    pltpu.async_copy(wts_h.at[sl0], wtsb.at[sl0], isems[0])

    plsc.subcore_barrier()

    def block_pair_body(q2, _):
      for par in range(2):
        q = q2 * 2 + par
        roff = par * BLK
        hoff = q * BLK
        for h, r in ((src_h, srcb), (dst_h, dstb), (wts_h, wtsb)):
            pltpu.make_async_copy(
                h.at[pl.ds(hoff, BLK)], r.at[pl.ds(roff, BLK)],
                isems[par]).wait()

        @pl.when(q > 0)
        def _():
            pltpu.make_async_copy(
                rows[LASTB], agg_s.at[dstb.at[BLK - 1 + (1 - par) * BLK]],
                ssems[LASTB]).wait()

        @pl.when(q + 1 < NBLK)
        def _():
            nroff = (1 - par) * BLK
            nhoff = hoff + BLK
            pltpu.async_copy(src_h.at[pl.ds(nhoff, BLK)],
                             srcb.at[pl.ds(nroff, BLK)], isems[1 - par])
            pltpu.async_copy(dst_h.at[pl.ds(nhoff, BLK)],
                             dstb.at[pl.ds(nroff, BLK)], isems[1 - par])
            pltpu.async_copy(wts_h.at[pl.ds(nhoff, BLK)],
                             wtsb.at[pl.ds(nroff, BLK)], isems[1 - par])

        for b in range(min(NBUF, BLK)):
            pltpu.async_copy(xs.at[srcb.at[roff + b]], rows[b], gsems[b])

        for t in range(BLK):
            b = t % NBUF
            bp = (t + NBUF - 1) % NBUF
            slot = roff + t
            pltpu.make_async_copy(
                xs.at[srcb.at[slot]], rows[b], gsems[b]).wait()

            if t >= 1:
                pltpu.make_async_copy(
                    rows[bp], agg_s.at[dstb.at[slot - 1]],
                    ssems[bp]).wait()
                if t + NBUF - 1 < BLK:
                    pltpu.async_copy(
                        xs.at[srcb.at[slot + NBUF - 1]], rows[bp],
                        gsems[bp])

            scale_rows(rows[b], slot)
            pltpu.async_copy(rows[b], agg_s.at[dstb.at[slot]], ssems[b],
                             add=True)
      return 0

    lax.fori_loop(0, NBLK // 2, block_pair_body, 0)

    pltpu.make_async_copy(
        rows[LASTB], agg_s.at[dstb.at[BLK - 1 + ((NBLK - 1) % 2) * BLK]],
        ssems[LASTB]).wait()

    plsc.subcore_barrier()

    off = 0
    for sz in _AGG_SLICES:
        pltpu.sync_copy(agg_s.at[pl.ds(base + off, sz)],
                        ph.at[pl.ds(base + off, sz)])
        off += sz


@functools.partial(
    pl.kernel,
    out_type=jax.ShapeDtypeStruct((NUM_CORES, N_PAD, HD), jnp.float32),
    mesh=plsc.VectorSubcoreMesh(core_axis_name="c", subcore_axis_name="s"),
    compiler_params=pltpu.CompilerParams(use_tc_tiling_on_sc=False),
    scratch_types=[
        pltpu.VMEM((2 * BLK, CHUNK), jnp.int32),
        pltpu.VMEM((2 * BLK, CHUNK), jnp.int32),
        pltpu.VMEM((2 * BLK, CHUNK), jnp.float32),
        [pltpu.VMEM((CHUNK, HD), jnp.float32)] * NBUF,
        [pltpu.SemaphoreType.DMA] * NBUF,
        [pltpu.SemaphoreType.DMA] * NBUF,
        [pltpu.SemaphoreType.DMA] * 2,
        pltpu.VMEM_SHARED((N_PAD, HD), jnp.float32),
        pltpu.VMEM_SHARED((N_PAD, HD), jnp.float32),
    ],
)
def _sc_aggregate(x_hbm, src_hbm, dst_hbm, wts_hbm, part_hbm,
                  srcb, dstb, wtsb, rows, gsems, ssems, isems, xs, agg_s):
    _sc_aggregate_body(x_hbm, src_hbm, dst_hbm, wts_hbm, part_hbm,
                       srcb, dstb, wtsb, rows, gsems, ssems, isems,
                       xs, agg_s)


def _tc_layer_body(p_ref, x_ref, w_ref, b_ref, o_ref):
    s = jnp.concatenate(
        [p_ref[0] + x_ref[0], p_ref[1] + x_ref[1]], axis=1)
    y = lax.dot_general(s, w_ref[...], (((1,), (1,)), ((), ())),
                        preferred_element_type=jnp.float32)
    y = jnp.maximum(y + b_ref[...], 0.0)
    o_ref[0] = y[:, :HD]
    o_ref[1] = y[:, HD:]


def _tc_final_body(p_ref, x_ref, w1_ref, b1_ref, wc1_ref, bc1_ref,
                   wc2_ref, bc2_ref, o_ref):
    s = jnp.concatenate(
        [p_ref[0] + x_ref[0], p_ref[1] + x_ref[1]], axis=1)
    x2 = lax.dot_general(s, w1_ref[...], (((1,), (1,)), ((), ())),
                         preferred_element_type=jnp.float32)
    x2 = jnp.maximum(x2 + b1_ref[...], 0.0)
    h = lax.dot_general(x2, wc1_ref[...], (((1,), (1,)), ((), ())),
                        preferred_element_type=jnp.float32)
    h = jnp.maximum(h + bc1_ref[...], 0.0)
    logits = lax.dot_general(h, wc2_ref[...], (((1,), (1,)), ((), ())),
                             preferred_element_type=jnp.float32)
    o_ref[...] = logits + bc2_ref[...]


_ROW_BLK = 2000


def _tc_layer(part, x, W, b):
    grid = (N_NODES_C // _ROW_BLK,)
    return pl.pallas_call(
        _tc_layer_body,
        grid=grid,
        in_specs=[
            pl.BlockSpec((NUM_CORES, _ROW_BLK, HD), lambda r: (0, r, 0)),
            pl.BlockSpec((NUM_CORES, _ROW_BLK, HD), lambda r: (0, r, 0)),
            pl.BlockSpec((D_C, D_C), lambda r: (0, 0)),
            pl.BlockSpec((1, D_C), lambda r: (0, 0)),
        ],
        out_specs=pl.BlockSpec((NUM_CORES, _ROW_BLK, HD), lambda r: (0, r, 0)),
        out_shape=jax.ShapeDtypeStruct((NUM_CORES, N_NODES_C, HD),
                                       jnp.float32),
    )(part, x, W, b)


def _tc_final(part, x, W1, b1, Wc1p, bc1p, Wc2p, bc2p):
    grid = (N_NODES_C // _ROW_BLK,)
    return pl.pallas_call(
        _tc_final_body,
        grid=grid,
        in_specs=[
            pl.BlockSpec((NUM_CORES, _ROW_BLK, HD), lambda r: (0, r, 0)),
            pl.BlockSpec((NUM_CORES, _ROW_BLK, HD), lambda r: (0, r, 0)),
            pl.BlockSpec((D_C, D_C), lambda r: (0, 0)),
            pl.BlockSpec((1, D_C), lambda r: (0, 0)),
            pl.BlockSpec((D_C, D_C), lambda r: (0, 0)),
            pl.BlockSpec((1, D_C), lambda r: (0, 0)),
            pl.BlockSpec((D_C, D_C), lambda r: (0, 0)),
            pl.BlockSpec((1, D_C), lambda r: (0, 0)),
        ],
        out_specs=pl.BlockSpec((_ROW_BLK, D_C), lambda r: (r, 0)),
        out_shape=jax.ShapeDtypeStruct((N_NODES_C, D_C), jnp.float32),
    )(part, x, W1, b1, Wc1p, bc1p, Wc2p, bc2p)


def kernel(features, edge_indices, edge_weights, W0, b0, W1, b1,
           Wc1, bc1, Wc2, bc2):
    ei = edge_indices[0].astype(jnp.int32)
    ew = edge_weights[0]
    pad = E_PAD - E_C
    src = jnp.concatenate([ei[0], jnp.zeros((pad,), jnp.int32)])
    dst = jnp.concatenate([ei[1], jnp.zeros((pad,), jnp.int32)])
    wts = jnp.concatenate([ew, jnp.zeros((pad,), jnp.float32)])
    src = src.reshape(NUM_SUBCORES, CHUNKS_PER_T, CHUNK)
    dst = dst.reshape(NUM_SUBCORES, CHUNKS_PER_T, CHUNK)
    wts = wts.reshape(NUM_SUBCORES, CHUNKS_PER_T, CHUNK)

    H2 = D_C // 2
    Wc1p = jnp.zeros((D_C, D_C), jnp.float32).at[:H2, :].set(Wc1)
    bc1p = jnp.zeros((1, D_C), jnp.float32).at[0, :H2].set(bc1)
    Wc2p = jnp.zeros((D_C, D_C), jnp.float32).at[:2, :H2].set(Wc2)
    bc2p = jnp.zeros((1, D_C), jnp.float32).at[0, :2].set(bc2)

    b0r = b0.reshape(1, D_C)
    b1r = b1.reshape(1, D_C)

    x = jnp.stack([features[:, :HD], features[:, HD:]])
    part0 = _sc_aggregate(x, src, dst, wts)
    x1 = _tc_layer(part0, x, W0, b0r)
    part1 = _sc_aggregate(x1, src, dst, wts)
    out = _tc_final(part1, x1, W1, b1r, Wc1p, bc1p, Wc2p, bc2p)
    return out[:, :2]
